# baseline (speedup 1.0000x reference)
import functools

import jax
import jax.numpy as jnp
from jax import lax
from jax.experimental import pallas as pl
from jax.experimental.pallas import tpu as pltpu
from jax.experimental.pallas import tpu_sc as plsc

D = 64
DP = 128
SCALE = 8.0
L = 16
NBUF = 4


def _build_sc_kernel(B: int, S: int, V: int):
    info = plsc.get_sparse_core_info()
    NW = info.num_cores * info.num_subcores
    CH = B // NW
    assert CH == DP and S % 4 == 0 and D % 8 == 0

    mesh = plsc.VectorSubcoreMesh(core_axis_name="c", subcore_axis_name="s")

    @functools.partial(
        pl.kernel,
        mesh=mesh,
        out_type=jax.ShapeDtypeStruct((S, D // 8, NW, 8, DP), jnp.float32),
        scratch_types=[
            pltpu.VMEM((S, CH), jnp.int32),
            *[pltpu.VMEM((CH, DP), jnp.float32) for _ in range(NBUF)],
            *[pltpu.VMEM((D // 8, 8, DP), jnp.float32) for _ in range(2)],
            *[pltpu.SemaphoreType.DMA for _ in range(NBUF + 2)],
        ],
        compiler_params=pltpu.CompilerParams(
            use_tc_tiling_on_sc=True, needs_layout_passes=False),
    )
    def k(xt_hbm, tp_hbm, out_hbm, idx_v, *refs):
        emb = refs[:NBUF]
        st = refs[NBUF:NBUF + 2]
        gsem = refs[NBUF + 2:2 * NBUF + 2]
        osem = refs[2 * NBUF + 2:]

        wid = lax.axis_index("s") * info.num_cores + lax.axis_index("c")

        pltpu.sync_copy(xt_hbm.at[pl.ds(0, S), pl.ds(wid * CH, CH)], idx_v)

        rowvecs = [lax.iota(jnp.int32, L) + 16 * kk for kk in range(CH // L)]

        def fire_gather(s, b):
            pltpu.async_copy(tp_hbm.at[idx_v.at[s]], emb[b], gsem[b])

        def drain_gather(b):
            pltpu.make_async_copy(tp_hbm.at[pl.ds(0, CH)], emb[b], gsem[b]).wait()

        def fire_out(s, p):
            for do in range(D // 8):
                pltpu.async_copy(st[p].at[do], out_hbm.at[s, do, wid], osem[p])

        def wait_out(p):
            for do in range(D // 8):
                pltpu.make_async_copy(
                    st[p].at[do], out_hbm.at[0, 0, 0], osem[p]).wait()

        def transpose_scale(b, p):
            def body(do, carry):
                for dd in range(8):
                    col = jnp.full((L,), 8 * do + dd, jnp.int32)
                    for kk in range(CH // L):
                        v = plsc.load_gather(emb[b], [rowvecs[kk], col])
                        st[p][do, dd, pl.ds(L * kk, L)] = v * SCALE
                return carry
            lax.fori_loop(0, D // 8, body, 0)

        def step(s, b, first, last):
            p = b % 2
            if not last:
                fire_gather(s + 2, (b + 2) % NBUF)
            drain_gather(b)
            if not first:
                wait_out(p)
            transpose_scale(b, p)
            fire_out(s, p)

        fire_gather(0, 0)
        fire_gather(1, 1)

        step(0, 0, first=True, last=False)
        step(1, 1, first=True, last=False)

        def rev(s4, carry):
            for c in range(4):
                step(2 + s4 * 4 + c, (2 + c) % NBUF, first=False, last=False)
            return carry
        lax.fori_loop(0, (S - 4) // 4, rev, 0)

        step(S - 2, (S - 2) % NBUF, first=False, last=True)
        step(S - 1, (S - 1) % NBUF, first=False, last=True)
        wait_out(0)
        wait_out(1)

    return k


def kernel(x, table):
    B, S = x.shape
    V = table.shape[0]
    xt = x.astype(jnp.int32).T
    tp = jnp.pad(table, ((0, 0), (0, DP - D)))
    out5 = _build_sc_kernel(B, S, V)(xt, tp)
    return out5.transpose(2, 4, 0, 1, 3).reshape(B, S, D)


# speedup vs baseline: 1.3026x; 1.3026x over previous
import functools

import jax
import jax.numpy as jnp
from jax import lax
from jax.experimental import pallas as pl
from jax.experimental.pallas import tpu as pltpu
from jax.experimental.pallas import tpu_sc as plsc

D = 64
DP = 128
SCALE = 8.0
L = 16
NBUF = 4


def _build_sc_kernel(B: int, S: int, V: int):
    info = plsc.get_sparse_core_info()
    NW = info.num_cores * info.num_subcores
    CH = B // NW
    assert CH == DP and S % 4 == 0 and D % 8 == 0

    mesh = plsc.VectorSubcoreMesh(core_axis_name="c", subcore_axis_name="s")

    @functools.partial(
        pl.kernel,
        mesh=mesh,
        out_type=jax.ShapeDtypeStruct((S, D // 8, NW, 8, DP), jnp.float32),
        scratch_types=[
            pltpu.VMEM((S, CH), jnp.int32),
            *[pltpu.VMEM((CH, DP), jnp.float32) for _ in range(NBUF)],
            *[pltpu.VMEM((D // 8, 8, DP), jnp.float32) for _ in range(2)],
            *[pltpu.SemaphoreType.DMA for _ in range(NBUF + 2)],
        ],
        compiler_params=pltpu.CompilerParams(
            use_tc_tiling_on_sc=True, needs_layout_passes=False),
    )
    def k(xt_hbm, tp_hbm, out_hbm, idx_v, *refs):
        emb = refs[:NBUF]
        st = refs[NBUF:NBUF + 2]
        gsem = refs[NBUF + 2:2 * NBUF + 2]
        osem = refs[2 * NBUF + 2:]

        wid = lax.axis_index("s") * info.num_cores + lax.axis_index("c")

        pltpu.sync_copy(xt_hbm.at[pl.ds(0, S), pl.ds(wid * CH, CH)], idx_v)

        rowvecs = [lax.iota(jnp.int32, L) + 16 * kk for kk in range(CH // L)]

        def fire_gather(s, b):
            pltpu.async_copy(tp_hbm.at[idx_v.at[s]], emb[b], gsem[b])

        def drain_gather(b):
            pltpu.make_async_copy(tp_hbm.at[pl.ds(0, CH)], emb[b], gsem[b]).wait()

        def fire_out(s, p):
            for do in range(D // 8):
                pltpu.async_copy(st[p].at[do], out_hbm.at[s, do, wid], osem[p])

        def wait_out(p):
            for do in range(D // 8):
                pltpu.make_async_copy(
                    st[p].at[do], out_hbm.at[0, 0, 0], osem[p]).wait()

        def transpose_scale(b, p):
            @plsc.parallel_loop(0, D // 8)
            def body(do):
                for dd in range(8):
                    col = jnp.full((L,), 8 * do + dd, jnp.int32)
                    vs = [plsc.load_gather(emb[b], [rowvecs[kk], col]) * SCALE
                          for kk in range(CH // L)]
                    for kk in range(CH // L):
                        st[p][do, dd, pl.ds(L * kk, L)] = vs[kk]

        def step(s, b, first, last):
            p = b % 2
            if not last:
                fire_gather(s + 2, (b + 2) % NBUF)
            drain_gather(b)
            if not first:
                wait_out(p)
            transpose_scale(b, p)
            fire_out(s, p)

        fire_gather(0, 0)
        fire_gather(1, 1)

        step(0, 0, first=True, last=False)
        step(1, 1, first=True, last=False)

        def rev(s4, carry):
            for c in range(4):
                step(2 + s4 * 4 + c, (2 + c) % NBUF, first=False, last=False)
            return carry
        lax.fori_loop(0, (S - 4) // 4, rev, 0)

        step(S - 2, (S - 2) % NBUF, first=False, last=True)
        step(S - 1, (S - 1) % NBUF, first=False, last=True)
        wait_out(0)
        wait_out(1)

    return k


def kernel(x, table):
    B, S = x.shape
    V = table.shape[0]
    xt = x.astype(jnp.int32).T
    tp = jnp.pad(table, ((0, 0), (0, DP - D)))
    out5 = _build_sc_kernel(B, S, V)(xt, tp)
    return out5.transpose(2, 4, 0, 1, 3).reshape(B, S, D)


# speedup vs baseline: 1.5252x; 1.1709x over previous
import functools

import jax
import jax.numpy as jnp
from jax import lax
from jax.experimental import pallas as pl
from jax.experimental.pallas import tpu as pltpu
from jax.experimental.pallas import tpu_sc as plsc

D = 64
DP = 128
SP = 136
SCALE = 8.0
L = 16
NBUF = 4


def _build_sc_kernel(B: int, S: int, V: int):
    info = plsc.get_sparse_core_info()
    NW = info.num_cores * info.num_subcores
    CH = B // NW
    assert CH == DP and S % 4 == 0 and D % 8 == 0

    mesh = plsc.VectorSubcoreMesh(core_axis_name="c", subcore_axis_name="s")

    @functools.partial(
        pl.kernel,
        mesh=mesh,
        out_type=jax.ShapeDtypeStruct((S, D // 8, NW, 8, DP), jnp.float32),
        scratch_types=[
            pltpu.VMEM((S, CH), jnp.int32),
            *[pltpu.VMEM((CH, DP), jnp.float32) for _ in range(NBUF)],
            *[pltpu.VMEM((D, SP), jnp.float32) for _ in range(2)],
            *[pltpu.SemaphoreType.DMA for _ in range(NBUF + 2)],
        ],
        compiler_params=pltpu.CompilerParams(
            use_tc_tiling_on_sc=True, needs_layout_passes=False),
    )
    def k(xt_hbm, tp_hbm, out_hbm, idx_v, *refs):
        emb = refs[:NBUF]
        st = refs[NBUF:NBUF + 2]
        gsem = refs[NBUF + 2:2 * NBUF + 2]
        osem = refs[2 * NBUF + 2:]

        wid = lax.axis_index("s") * info.num_cores + lax.axis_index("c")

        pltpu.sync_copy(xt_hbm.at[pl.ds(0, S), pl.ds(wid * CH, CH)], idx_v)

        dvecs = [lax.iota(jnp.int32, L) + L * kk for kk in range(D // L)]

        def fire_gather(s, b):
            pltpu.async_copy(tp_hbm.at[idx_v.at[s]], emb[b], gsem[b])

        def drain_gather(b):
            pltpu.make_async_copy(tp_hbm.at[pl.ds(0, CH)], emb[b], gsem[b]).wait()

        def fire_out(s, p):
            for do in range(D // 8):
                pltpu.async_copy(
                    st[p].at[pl.ds(8 * do, 8), pl.ds(0, DP)],
                    out_hbm.at[s, do, wid], osem[p])

        def wait_out(p):
            for do in range(D // 8):
                pltpu.make_async_copy(
                    st[p].at[pl.ds(0, 8), pl.ds(0, DP)],
                    out_hbm.at[0, 0, 0], osem[p]).wait()

        def transpose_scale(b, p):
            @plsc.parallel_loop(0, CH, unroll=2)
            def body(bb):
                bbs = jnp.full((L,), bb, jnp.int32)
                for kk in range(D // L):
                    v = emb[b][bb, pl.ds(L * kk, L)] * SCALE
                    plsc.store_scatter(st[p], [dvecs[kk], bbs], v)

        def step(s, b, first, last):
            p = b % 2
            if not last:
                fire_gather(s + 2, (b + 2) % NBUF)
            drain_gather(b)
            if not first:
                wait_out(p)
            transpose_scale(b, p)
            fire_out(s, p)

        fire_gather(0, 0)
        fire_gather(1, 1)

        step(0, 0, first=True, last=False)
        step(1, 1, first=True, last=False)

        def rev(s4, carry):
            for c in range(4):
                step(2 + s4 * 4 + c, (2 + c) % NBUF, first=False, last=False)
            return carry
        lax.fori_loop(0, (S - 4) // 4, rev, 0)

        step(S - 2, (S - 2) % NBUF, first=False, last=True)
        step(S - 1, (S - 1) % NBUF, first=False, last=True)
        wait_out(0)
        wait_out(1)

    return k


def kernel(x, table):
    B, S = x.shape
    V = table.shape[0]
    xt = x.astype(jnp.int32).T
    tp = jnp.pad(table, ((0, 0), (0, DP - D)))
    out5 = _build_sc_kernel(B, S, V)(xt, tp)
    return out5.transpose(2, 4, 0, 1, 3).reshape(B, S, D)


# speedup vs baseline: 2.7481x; 1.8017x over previous
import functools

import jax
import jax.numpy as jnp
from jax import lax
from jax.experimental import pallas as pl
from jax.experimental.pallas import tpu as pltpu
from jax.experimental.pallas import tpu_sc as plsc

D = 64
DP = 128
SP = 136
SCALE = 8.0
L = 16
NBUF = 8
NST = 4


def _build_sc_kernel(B: int, S: int, V: int):
    info = plsc.get_sparse_core_info()
    NW = info.num_cores * info.num_subcores
    CH = B // NW
    assert CH == 128 and S % NBUF == 0 and D % L == 0

    mesh = plsc.VectorSubcoreMesh(core_axis_name="c", subcore_axis_name="s")

    @functools.partial(
        pl.kernel,
        mesh=mesh,
        out_type=jax.ShapeDtypeStruct((S, D // 8, NW, 8, DP), jnp.float32),
        scratch_types=[
            pltpu.VMEM((S, CH), jnp.int32),
            *[pltpu.VMEM((CH, D), jnp.float32) for _ in range(NBUF)],
            *[pltpu.VMEM((D, SP), jnp.float32) for _ in range(NST)],
            *[pltpu.SemaphoreType.DMA for _ in range(NBUF + NST)],
        ],
        compiler_params=pltpu.CompilerParams(
            use_tc_tiling_on_sc=False, needs_layout_passes=False),
    )
    def k(xt_hbm, tp_hbm, out_hbm, idx_v, *refs):
        emb = refs[:NBUF]
        st = refs[NBUF:NBUF + NST]
        gsem = refs[NBUF + NST:2 * NBUF + NST]
        osem = refs[2 * NBUF + NST:]

        wid = lax.axis_index("s") * info.num_cores + lax.axis_index("c")

        pltpu.sync_copy(xt_hbm.at[pl.ds(0, S), pl.ds(wid * CH, CH)], idx_v)

        dvecs = [lax.iota(jnp.int32, L) + L * kk for kk in range(D // L)]

        def fire_gather(s, b):
            pltpu.async_copy(tp_hbm.at[idx_v.at[s]], emb[b], gsem[b])

        def drain_gather(b):
            pltpu.make_async_copy(tp_hbm.at[pl.ds(0, CH)], emb[b], gsem[b]).wait()

        def fire_out(s, p):
            for do in range(D // 8):
                pltpu.async_copy(
                    st[p].at[pl.ds(8 * do, 8), pl.ds(0, DP)],
                    out_hbm.at[s, do, wid], osem[p])

        def wait_out(p):
            for do in range(D // 8):
                pltpu.make_async_copy(
                    st[p].at[pl.ds(0, 8), pl.ds(0, DP)],
                    out_hbm.at[0, 0, 0], osem[p]).wait()

        def transpose_scale(b, p):
            @plsc.parallel_loop(0, CH, unroll=2)
            def body(bb):
                bbs = jnp.full((L,), bb, jnp.int32)
                for kk in range(D // L):
                    v = emb[b][bb, pl.ds(L * kk, L)] * SCALE
                    plsc.store_scatter(st[p], [dvecs[kk], bbs], v)

        def step(s, u, first, last):
            b, p = u % NBUF, u % NST
            if not last:
                fire_gather(s + 6, (u + 6) % NBUF)
            drain_gather(b)
            if not first:
                wait_out(p)
            transpose_scale(b, p)
            fire_out(s, p)

        for s in range(6):
            fire_gather(s, s)

        for u in range(NBUF):
            step(u, u, first=(u < NST), last=False)

        def rev(s8, carry):
            for u in range(NBUF):
                step(s8 * NBUF + u, u, first=False, last=False)
            return carry
        lax.fori_loop(1, S // NBUF - 1, rev, 0)

        base = (S // NBUF - 1) * NBUF
        for u in range(NBUF):
            step(base + u, u, first=False, last=(u >= 2))
        for p in range(NST):
            wait_out(p)

    return k


def kernel(x, table):
    B, S = x.shape
    V = table.shape[0]
    xt = (x.astype(jnp.int32) * 2).T
    tp = jnp.pad(table, ((0, 0), (0, DP - D))).reshape(2 * V, D)
    out5 = _build_sc_kernel(B, S, V)(xt, tp)
    return out5.transpose(2, 4, 0, 1, 3).reshape(B, S, D)
